# Initial kernel scaffold; baseline (speedup 1.0000x reference)
import jax
import jax.numpy as jnp
from jax.experimental import pallas as pl


def kernel(features, edge_index, att, W, b, Wl, bl, Wr, br):
    raise NotImplementedError("write your pallas kernel here")


# speedup vs baseline: 38.9533x; 38.9533x over previous
import functools

import jax
import jax.numpy as jnp
from jax import lax
from jax.experimental import pallas as pl
from jax.experimental.pallas import tpu as pltpu
from jax.experimental.pallas import tpu_sc as plsc

NUM_GRAPH = 4
HID = 32
SIGMA = 1.0

NC = 2
NS = 16
NW = NC * NS


def _tc_dense_kernel(f_ref, att_ref, w_ref, b_ref, wl_ref, bl_ref,
                     wr_ref, br_ref, hid_ref, al_ref, ar_ref):
    f = f_ref[...]
    for g in range(NUM_GRAPH):
        fa = f * att_ref[g, :][None, :]
        h = jnp.dot(fa, w_ref[g], preferred_element_type=jnp.float32)
        h = h + b_ref[g, :][None, :]
        hid_ref[:, g * HID:(g + 1) * HID] = h
        al_ref[:, g:g + 1] = (
            jnp.dot(h, wl_ref[g], preferred_element_type=jnp.float32)
            + bl_ref[g][None, :])
        ar_ref[:, g:g + 1] = (
            jnp.dot(h, wr_ref[g], preferred_element_type=jnp.float32)
            + br_ref[g][None, :])


def _dense_part(features, att, W, b, Wl, bl, Wr, br):
    n, d = features.shape
    blk = 1000
    grid = n // blk
    full = lambda *dims: pl.BlockSpec(dims, lambda i: (0,) * len(dims))
    return pl.pallas_call(
        _tc_dense_kernel,
        grid=(grid,),
        in_specs=[
            pl.BlockSpec((blk, d), lambda i: (i, 0)),
            full(NUM_GRAPH, d),
            full(NUM_GRAPH, d, HID),
            full(NUM_GRAPH, HID),
            full(NUM_GRAPH, HID, 1),
            full(NUM_GRAPH, 1),
            full(NUM_GRAPH, HID, 1),
            full(NUM_GRAPH, 1),
        ],
        out_specs=[
            pl.BlockSpec((blk, NUM_GRAPH * HID), lambda i: (i, 0)),
            pl.BlockSpec((blk, NUM_GRAPH), lambda i: (i, 0)),
            pl.BlockSpec((blk, NUM_GRAPH), lambda i: (i, 0)),
        ],
        out_shape=[
            jax.ShapeDtypeStruct((n, NUM_GRAPH * HID), jnp.float32),
            jax.ShapeDtypeStruct((n, NUM_GRAPH), jnp.float32),
            jax.ShapeDtypeStruct((n, NUM_GRAPH), jnp.float32),
        ],
    )(features, att, W, b, Wl, bl, Wr, br)


def _make_edge_kernel(n4, e, ch):
    epw = e // NW
    nch = epw // ch
    mesh = plsc.VectorSubcoreMesh(core_axis_name="c", subcore_axis_name="s")

    @functools.partial(
        pl.kernel, mesh=mesh,
        compiler_params=pltpu.CompilerParams(needs_layout_passes=False),
        out_type=jax.ShapeDtypeStruct((e * NUM_GRAPH,), jnp.float32),
        scratch_types=[
            pltpu.VMEM((n4,), jnp.float32),
            pltpu.VMEM((n4,), jnp.float32),
            pltpu.VMEM((ch,), jnp.int32),
            pltpu.VMEM((ch,), jnp.int32),
            pltpu.VMEM((ch * NUM_GRAPH,), jnp.float32),
        ],
    )
    def edge_kernel(al_hbm, ar_hbm, src_hbm, dst_hbm, out_hbm,
                    al_v, ar_v, src_v, dst_v, out_v):
        wid = lax.axis_index("s") * NC + lax.axis_index("c")
        pltpu.sync_copy(al_hbm, al_v)
        pltpu.sync_copy(ar_hbm, ar_v)
        lane = lax.iota(jnp.int32, 16)
        base0 = wid * epw
        for c in range(nch):
            base = base0 + c * ch
            pltpu.sync_copy(src_hbm.at[pl.ds(base, ch)], src_v)
            pltpu.sync_copy(dst_hbm.at[pl.ds(base, ch)], dst_v)

            def body(g16, _):
                s = src_v[pl.ds(g16 * 16, 16)] * NUM_GRAPH
                d = dst_v[pl.ds(g16 * 16, 16)] * NUM_GRAPH
                obase = g16 * (16 * NUM_GRAPH) + lane * NUM_GRAPH
                for g in range(NUM_GRAPH):
                    av = plsc.load_gather(al_v, [s + g])
                    rv = plsc.load_gather(ar_v, [d + g])
                    x = av + rv
                    sig = 1.0 / (1.0 + jnp.exp(-x))
                    plsc.store_scatter(out_v, [obase + g], sig)
                return 0

            lax.fori_loop(0, ch // 16, body, 0)
            pltpu.sync_copy(
                out_v, out_hbm.at[pl.ds(base * NUM_GRAPH, ch * NUM_GRAPH)])

    return edge_kernel


def kernel(features, edge_index, att, W, b, Wl, bl, Wr, br):
    n = features.shape[0]
    e = edge_index.shape[1]
    hidden, al, ar = _dense_part(features, att, W, b, Wl, bl, Wr, br)
    alf = al.reshape(-1)
    arf = ar.reshape(-1)
    src = edge_index[0]
    dst = edge_index[1]
    edge_kernel = _make_edge_kernel(n * NUM_GRAPH, e, 2000)
    factors_flat = edge_kernel(alf, arf, src, dst)
    factors = factors_flat.reshape(e, NUM_GRAPH)
    return hidden, factors


# speedup vs baseline: 47.4285x; 1.2176x over previous
import functools

import jax
import jax.numpy as jnp
from jax import lax
from jax.experimental import pallas as pl
from jax.experimental.pallas import tpu as pltpu
from jax.experimental.pallas import tpu_sc as plsc

NUM_GRAPH = 4
HID = 32
SIGMA = 1.0

NC = 2
NS = 16
NW = NC * NS


def _tc_dense_kernel(f_ref, att_ref, w_ref, b_ref, wl_ref, bl_ref,
                     wr_ref, br_ref, hid_ref, alr_ref):
    f = f_ref[...]
    for g in range(NUM_GRAPH):
        fa = f * att_ref[g, :][None, :]
        h = jnp.dot(fa, w_ref[g], preferred_element_type=jnp.float32)
        h = h + b_ref[g, :][None, :]
        hid_ref[:, g * HID:(g + 1) * HID] = h
        alr_ref[:, g:g + 1] = (
            jnp.dot(h, wl_ref[g], preferred_element_type=jnp.float32)
            + bl_ref[g][None, :])
        alr_ref[:, NUM_GRAPH + g:NUM_GRAPH + g + 1] = (
            jnp.dot(h, wr_ref[g], preferred_element_type=jnp.float32)
            + br_ref[g][None, :])


def _dense_part(features, att, W, b, Wl, bl, Wr, br):
    n, d = features.shape
    blk = 1000
    grid = n // blk
    full = lambda *dims: pl.BlockSpec(dims, lambda i: (0,) * len(dims))
    return pl.pallas_call(
        _tc_dense_kernel,
        grid=(grid,),
        in_specs=[
            pl.BlockSpec((blk, d), lambda i: (i, 0)),
            full(NUM_GRAPH, d),
            full(NUM_GRAPH, d, HID),
            full(NUM_GRAPH, HID),
            full(NUM_GRAPH, HID, 1),
            full(NUM_GRAPH, 1),
            full(NUM_GRAPH, HID, 1),
            full(NUM_GRAPH, 1),
        ],
        out_specs=[
            pl.BlockSpec((blk, NUM_GRAPH * HID), lambda i: (i, 0)),
            pl.BlockSpec((blk, 2 * NUM_GRAPH), lambda i: (i, 0)),
        ],
        out_shape=[
            jax.ShapeDtypeStruct((n, NUM_GRAPH * HID), jnp.float32),
            jax.ShapeDtypeStruct((n, 2 * NUM_GRAPH), jnp.float32),
        ],
    )(features, att, W, b, Wl, bl, Wr, br)


def _make_edge_kernel(n, e, ch):
    epw = e // NW
    nch = epw // ch
    mesh = plsc.VectorSubcoreMesh(core_axis_name="c", subcore_axis_name="s")

    @functools.partial(
        pl.kernel, mesh=mesh,
        compiler_params=pltpu.CompilerParams(
            needs_layout_passes=False, use_tc_tiling_on_sc=False),
        out_type=jax.ShapeDtypeStruct((e, NUM_GRAPH), jnp.float32),
        scratch_types=[
            pltpu.VMEM((n, 2 * NUM_GRAPH), jnp.float32),
            pltpu.VMEM((ch,), jnp.int32),
            pltpu.VMEM((ch,), jnp.int32),
            pltpu.VMEM((ch, NUM_GRAPH), jnp.float32),
        ],
    )
    def edge_kernel(alr_hbm, src_hbm, dst_hbm, out_hbm, alr_v, src_v, dst_v,
                    out_v):
        wid = lax.axis_index("s") * NC + lax.axis_index("c")
        pltpu.sync_copy(alr_hbm, alr_v)
        lane = lax.iota(jnp.int32, 16)
        cols = [jnp.full((16,), g, dtype=jnp.int32) for g in range(2 * NUM_GRAPH)]
        base0 = wid * epw
        for c in range(nch):
            base = base0 + c * ch
            pltpu.sync_copy(src_hbm.at[pl.ds(base, ch)], src_v)
            pltpu.sync_copy(dst_hbm.at[pl.ds(base, ch)], dst_v)

            def body(g16, _):
                s = src_v[pl.ds(g16 * 16, 16)]
                d = dst_v[pl.ds(g16 * 16, 16)]
                rows = g16 * 16 + lane
                for g in range(NUM_GRAPH):
                    av = plsc.load_gather(alr_v, [s, cols[g]])
                    rv = plsc.load_gather(alr_v, [d, cols[NUM_GRAPH + g]])
                    x = av + rv
                    sig = 1.0 / (1.0 + jnp.exp(-x))
                    plsc.store_scatter(out_v, [rows, cols[g]], sig)
                return 0

            lax.fori_loop(0, ch // 16, body, 0)
            pltpu.sync_copy(out_v, out_hbm.at[pl.ds(base, ch)])

    return edge_kernel


def kernel(features, edge_index, att, W, b, Wl, bl, Wr, br):
    n = features.shape[0]
    e = edge_index.shape[1]
    hidden, alr = _dense_part(features, att, W, b, Wl, bl, Wr, br)
    edge_kernel = _make_edge_kernel(n, e, 2000)
    factors = edge_kernel(alr, edge_index[0], edge_index[1])
    return hidden, factors


# speedup vs baseline: 151.5726x; 3.1958x over previous
import functools

import jax
import jax.numpy as jnp
from jax import lax
from jax.experimental import pallas as pl
from jax.experimental.pallas import tpu as pltpu
from jax.experimental.pallas import tpu_sc as plsc

NUM_GRAPH = 4
HID = 32
SIGMA = 1.0

NC = 2
NS = 16
NW = NC * NS


def _tc_dense_kernel(f_ref, att_ref, w_ref, b_ref, wl_ref, bl_ref,
                     wr_ref, br_ref, hid_ref, alr_ref):
    f = f_ref[...]
    for g in range(NUM_GRAPH):
        fa = f * att_ref[g, :][None, :]
        h = jnp.dot(fa, w_ref[g], preferred_element_type=jnp.float32)
        h = h + b_ref[g, :][None, :]
        hid_ref[:, g * HID:(g + 1) * HID] = h
        alr_ref[:, g:g + 1] = (
            jnp.dot(h, wl_ref[g], preferred_element_type=jnp.float32)
            + bl_ref[g][None, :])
        alr_ref[:, NUM_GRAPH + g:NUM_GRAPH + g + 1] = (
            jnp.dot(h, wr_ref[g], preferred_element_type=jnp.float32)
            + br_ref[g][None, :])


def _dense_part(features, att, W, b, Wl, bl, Wr, br):
    n, d = features.shape
    blk = 1000
    grid = n // blk
    full = lambda *dims: pl.BlockSpec(dims, lambda i: (0,) * len(dims))
    return pl.pallas_call(
        _tc_dense_kernel,
        grid=(grid,),
        in_specs=[
            pl.BlockSpec((blk, d), lambda i: (i, 0)),
            full(NUM_GRAPH, d),
            full(NUM_GRAPH, d, HID),
            full(NUM_GRAPH, HID),
            full(NUM_GRAPH, HID, 1),
            full(NUM_GRAPH, 1),
            full(NUM_GRAPH, HID, 1),
            full(NUM_GRAPH, 1),
        ],
        out_specs=[
            pl.BlockSpec((blk, NUM_GRAPH * HID), lambda i: (i, 0)),
            pl.BlockSpec((blk, 2 * NUM_GRAPH), lambda i: (i, 0)),
        ],
        out_shape=[
            jax.ShapeDtypeStruct((n, NUM_GRAPH * HID), jnp.float32),
            jax.ShapeDtypeStruct((n, 2 * NUM_GRAPH), jnp.float32),
        ],
    )(features, att, W, b, Wl, bl, Wr, br)


def _make_edge_kernel(n, e, ch):
    epw = e // NW
    nch = epw // ch
    tw = 2 * NUM_GRAPH
    mesh = plsc.VectorSubcoreMesh(core_axis_name="c", subcore_axis_name="s")

    @functools.partial(
        pl.kernel, mesh=mesh,
        compiler_params=pltpu.CompilerParams(
            needs_layout_passes=False, use_tc_tiling_on_sc=False),
        out_type=jax.ShapeDtypeStruct((NUM_GRAPH, e), jnp.float32),
        scratch_types=[
            pltpu.VMEM((n * tw,), jnp.float32),
            pltpu.VMEM((ch,), jnp.int32),
            pltpu.VMEM((ch,), jnp.int32),
            pltpu.VMEM((NUM_GRAPH * ch,), jnp.float32),
        ],
    )
    def edge_kernel(alr_hbm, src_hbm, dst_hbm, out_hbm, alr_v, src_v, dst_v,
                    out_v):
        wid = lax.axis_index("s") * NC + lax.axis_index("c")
        pltpu.sync_copy(alr_hbm, alr_v)
        base0 = wid * epw
        for c in range(nch):
            base = base0 + c * ch
            pltpu.sync_copy(src_hbm.at[pl.ds(base, ch)], src_v)
            pltpu.sync_copy(dst_hbm.at[pl.ds(base, ch)], dst_v)

            @plsc.parallel_loop(0, ch, 16, unroll=4)
            def body(i):
                s = src_v[pl.ds(i, 16)] * tw
                d = dst_v[pl.ds(i, 16)] * tw + NUM_GRAPH
                for g in range(NUM_GRAPH):
                    av = plsc.load_gather(alr_v, [s + g])
                    rv = plsc.load_gather(alr_v, [d + g])
                    x = av + rv
                    out_v[pl.ds(g * ch + i, 16)] = 1.0 / (1.0 + jnp.exp(-x))

            for g in range(NUM_GRAPH):
                pltpu.sync_copy(out_v.at[pl.ds(g * ch, ch)],
                                out_hbm.at[g, pl.ds(base, ch)])

    return edge_kernel


def kernel(features, edge_index, att, W, b, Wl, bl, Wr, br):
    n = features.shape[0]
    e = edge_index.shape[1]
    hidden, alr = _dense_part(features, att, W, b, Wl, bl, Wr, br)
    edge_kernel = _make_edge_kernel(n, e, 2000)
    factors_t = edge_kernel(alr.reshape(-1), edge_index[0], edge_index[1])
    return hidden, factors_t.T


# speedup vs baseline: 151.9300x; 1.0024x over previous
import functools

import jax
import jax.numpy as jnp
from jax import lax
from jax.experimental import pallas as pl
from jax.experimental.pallas import tpu as pltpu
from jax.experimental.pallas import tpu_sc as plsc

NUM_GRAPH = 4
HID = 32
SIGMA = 1.0

NC = 2
NS = 16
NW = NC * NS


def _tc_dense_kernel(f_ref, att_ref, w_ref, b_ref, wl_ref, bl_ref,
                     wr_ref, br_ref, hid_ref, alr_ref):
    f = f_ref[...]
    for g in range(NUM_GRAPH):
        fa = f * att_ref[g, :][None, :]
        h = jnp.dot(fa, w_ref[g], preferred_element_type=jnp.float32)
        h = h + b_ref[g, :][None, :]
        hid_ref[:, g * HID:(g + 1) * HID] = h
        alr_ref[:, g:g + 1] = (
            jnp.dot(h, wl_ref[g], preferred_element_type=jnp.float32)
            + bl_ref[g][None, :])
        alr_ref[:, NUM_GRAPH + g:NUM_GRAPH + g + 1] = (
            jnp.dot(h, wr_ref[g], preferred_element_type=jnp.float32)
            + br_ref[g][None, :])


def _dense_part(features, att, W, b, Wl, bl, Wr, br):
    n, d = features.shape
    blk = 1000
    grid = n // blk
    full = lambda *dims: pl.BlockSpec(dims, lambda i: (0,) * len(dims))
    return pl.pallas_call(
        _tc_dense_kernel,
        grid=(grid,),
        in_specs=[
            pl.BlockSpec((blk, d), lambda i: (i, 0)),
            full(NUM_GRAPH, d),
            full(NUM_GRAPH, d, HID),
            full(NUM_GRAPH, HID),
            full(NUM_GRAPH, HID, 1),
            full(NUM_GRAPH, 1),
            full(NUM_GRAPH, HID, 1),
            full(NUM_GRAPH, 1),
        ],
        out_specs=[
            pl.BlockSpec((blk, NUM_GRAPH * HID), lambda i: (i, 0)),
            pl.BlockSpec((blk, 2 * NUM_GRAPH), lambda i: (i, 0)),
        ],
        out_shape=[
            jax.ShapeDtypeStruct((n, NUM_GRAPH * HID), jnp.float32),
            jax.ShapeDtypeStruct((n, 2 * NUM_GRAPH), jnp.float32),
        ],
    )(features, att, W, b, Wl, bl, Wr, br)


def _make_edge_kernel(n, e, ch):
    epw = e // NW
    nch = epw // ch
    tw = 2 * NUM_GRAPH
    mesh = plsc.VectorSubcoreMesh(core_axis_name="c", subcore_axis_name="s")

    @functools.partial(
        pl.kernel, mesh=mesh,
        compiler_params=pltpu.CompilerParams(
            needs_layout_passes=False, use_tc_tiling_on_sc=False),
        out_type=jax.ShapeDtypeStruct((NUM_GRAPH, e), jnp.float32),
        scratch_types=[
            pltpu.VMEM((n * tw,), jnp.float32),
            pltpu.VMEM((epw,), jnp.int32),
            pltpu.VMEM((epw,), jnp.int32),
            pltpu.VMEM((2 * NUM_GRAPH * ch,), jnp.float32),
            pltpu.SemaphoreType.DMA,
            pltpu.SemaphoreType.DMA,
        ],
    )
    def edge_kernel(alr_hbm, src_hbm, dst_hbm, out_hbm, alr_v, src_v, dst_v,
                    out_v, sem_in, sem_out):
        wid = lax.axis_index("s") * NC + lax.axis_index("c")
        base0 = wid * epw
        h_tab = pltpu.async_copy(alr_hbm, alr_v, sem_in)
        h_src = pltpu.async_copy(src_hbm.at[pl.ds(base0, epw)], src_v, sem_in)
        h_dst = pltpu.async_copy(dst_hbm.at[pl.ds(base0, epw)], dst_v, sem_in)
        h_tab.wait()
        h_src.wait()
        h_dst.wait()
        out_handles = {}
        for c in range(nch):
            buf = (c % 2) * NUM_GRAPH * ch
            if c >= 2:
                for h in out_handles.pop(c - 2):
                    h.wait()

            @plsc.parallel_loop(0, ch, 16, unroll=8)
            def body(i):
                s = src_v[pl.ds(c * ch + i, 16)] * tw
                d = dst_v[pl.ds(c * ch + i, 16)] * tw + NUM_GRAPH
                for g in range(NUM_GRAPH):
                    av = plsc.load_gather(alr_v, [s + g])
                    rv = plsc.load_gather(alr_v, [d + g])
                    x = av + rv
                    out_v[pl.ds(buf + g * ch + i, 16)] = (
                        1.0 / (1.0 + jnp.exp(-x)))

            out_handles[c] = [
                pltpu.async_copy(out_v.at[pl.ds(buf + g * ch, ch)],
                                 out_hbm.at[g, pl.ds(base0 + c * ch, ch)],
                                 sem_out)
                for g in range(NUM_GRAPH)]
        for hs in out_handles.values():
            for h in hs:
                h.wait()

    return edge_kernel


def kernel(features, edge_index, att, W, b, Wl, bl, Wr, br):
    n = features.shape[0]
    e = edge_index.shape[1]
    hidden, alr = _dense_part(features, att, W, b, Wl, bl, Wr, br)
    edge_kernel = _make_edge_kernel(n, e, 2000)
    factors_t = edge_kernel(alr.reshape(-1), edge_index[0], edge_index[1])
    return hidden, factors_t.T


# speedup vs baseline: 157.7404x; 1.0382x over previous
import functools

import jax
import jax.numpy as jnp
from jax import lax
from jax.experimental import pallas as pl
from jax.experimental.pallas import tpu as pltpu
from jax.experimental.pallas import tpu_sc as plsc

NUM_GRAPH = 4
HID = 32
SIGMA = 1.0
TW = 2 * NUM_GRAPH + 1

NC = 2
NS = 16
NW = NC * NS


def _tc_dense_kernel(f_ref, att_ref, w_ref, b_ref, wl_ref, bl_ref,
                     wr_ref, br_ref, hid_ref, alr_ref):
    f = f_ref[...]
    alr_ref[:, 2 * NUM_GRAPH:] = jnp.zeros_like(alr_ref[:, 2 * NUM_GRAPH:])
    for g in range(NUM_GRAPH):
        fa = f * att_ref[g, :][None, :]
        h = jnp.dot(fa, w_ref[g], preferred_element_type=jnp.float32)
        h = h + b_ref[g, :][None, :]
        hid_ref[:, g * HID:(g + 1) * HID] = h
        alr_ref[:, g:g + 1] = (
            jnp.dot(h, wl_ref[g], preferred_element_type=jnp.float32)
            + bl_ref[g][None, :])
        alr_ref[:, NUM_GRAPH + g:NUM_GRAPH + g + 1] = (
            jnp.dot(h, wr_ref[g], preferred_element_type=jnp.float32)
            + br_ref[g][None, :])


def _dense_part(features, att, W, b, Wl, bl, Wr, br):
    n, d = features.shape
    blk = 1000
    grid = n // blk
    full = lambda *dims: pl.BlockSpec(dims, lambda i: (0,) * len(dims))
    return pl.pallas_call(
        _tc_dense_kernel,
        grid=(grid,),
        in_specs=[
            pl.BlockSpec((blk, d), lambda i: (i, 0)),
            full(NUM_GRAPH, d),
            full(NUM_GRAPH, d, HID),
            full(NUM_GRAPH, HID),
            full(NUM_GRAPH, HID, 1),
            full(NUM_GRAPH, 1),
            full(NUM_GRAPH, HID, 1),
            full(NUM_GRAPH, 1),
        ],
        out_specs=[
            pl.BlockSpec((blk, NUM_GRAPH * HID), lambda i: (i, 0)),
            pl.BlockSpec((blk, TW), lambda i: (i, 0)),
        ],
        out_shape=[
            jax.ShapeDtypeStruct((n, NUM_GRAPH * HID), jnp.float32),
            jax.ShapeDtypeStruct((n, TW), jnp.float32),
        ],
    )(features, att, W, b, Wl, bl, Wr, br)


def _make_edge_kernel(n, e, ch):
    epw = e // NW
    nch = epw // ch
    mesh = plsc.VectorSubcoreMesh(core_axis_name="c", subcore_axis_name="s")

    @functools.partial(
        pl.kernel, mesh=mesh,
        compiler_params=pltpu.CompilerParams(
            needs_layout_passes=False, use_tc_tiling_on_sc=False),
        out_type=jax.ShapeDtypeStruct((NUM_GRAPH, e), jnp.float32),
        scratch_types=[
            pltpu.VMEM((n * TW,), jnp.float32),
            pltpu.VMEM((epw,), jnp.int32),
            pltpu.VMEM((epw,), jnp.int32),
            pltpu.VMEM((2 * NUM_GRAPH * ch,), jnp.float32),
            pltpu.SemaphoreType.DMA,
            pltpu.SemaphoreType.DMA,
        ],
    )
    def edge_kernel(alr_hbm, src_hbm, dst_hbm, out_hbm, alr_v, src_v, dst_v,
                    out_v, sem_in, sem_out):
        wid = lax.axis_index("s") * NC + lax.axis_index("c")
        base0 = wid * epw
        h_tab = pltpu.async_copy(alr_hbm, alr_v, sem_in)
        h_src = pltpu.async_copy(src_hbm.at[pl.ds(base0, epw)], src_v, sem_in)
        h_dst = pltpu.async_copy(dst_hbm.at[pl.ds(base0, epw)], dst_v, sem_in)
        h_tab.wait()
        h_src.wait()
        h_dst.wait()
        out_handles = {}
        for c in range(nch):
            buf = (c % 2) * NUM_GRAPH * ch
            if c >= 2:
                for h in out_handles.pop(c - 2):
                    h.wait()

            @plsc.parallel_loop(0, ch, 16, unroll=8)
            def body(i):
                s = src_v[pl.ds(c * ch + i, 16)] * TW
                d = dst_v[pl.ds(c * ch + i, 16)] * TW + NUM_GRAPH
                for g in range(NUM_GRAPH):
                    av = plsc.load_gather(alr_v, [s + g])
                    rv = plsc.load_gather(alr_v, [d + g])
                    x = av + rv
                    out_v[pl.ds(buf + g * ch + i, 16)] = (
                        1.0 / (1.0 + jnp.exp(-x)))

            out_handles[c] = [
                pltpu.async_copy(out_v.at[pl.ds(buf + g * ch, ch)],
                                 out_hbm.at[g, pl.ds(base0 + c * ch, ch)],
                                 sem_out)
                for g in range(NUM_GRAPH)]
        for hs in out_handles.values():
            for h in hs:
                h.wait()

    return edge_kernel


def kernel(features, edge_index, att, W, b, Wl, bl, Wr, br):
    n = features.shape[0]
    e = edge_index.shape[1]
    hidden, alr = _dense_part(features, att, W, b, Wl, bl, Wr, br)
    edge_kernel = _make_edge_kernel(n, e, 2000)
    factors_t = edge_kernel(alr.reshape(-1), edge_index[0], edge_index[1])
    return hidden, factors_t.T
